# baseline (device time: 117959 ns/iter reference)
import jax
import jax.numpy as jnp
from jax import lax
from jax.experimental import pallas as pl
from jax.experimental.pallas import tpu as pltpu

N_DEV = 4
SQ = 2048
SKV = 2048
HQ = 8
DH = 128
DM = HQ * DH
SCALE = 0.08838834764831843
QBLK = 128
N_QBLK = SQ // QBLK
PACK = DM + 128
SLAB = 4 * QBLK
N_CHUNK = 8
CHUNK = SQ // N_CHUNK
HALF = N_CHUNK // 2
TINY = 160

SEGS = ((1, 6, (1, 2)), (6, 11, (3, 4)), (11, 15, (5, 6)))
ORDER = (1, 2, 3, 4, 5, 6, 0, 7)


def kernel(x, Wq, K_ext, V_ext, Wo):
    xb = x.reshape(SQ, DM).astype(jnp.bfloat16)
    Kb = K_ext.reshape(SKV, DM).astype(jnp.bfloat16)
    Vb = V_ext.reshape(SKV, DM).astype(jnp.bfloat16)
    Wqb = Wq.astype(jnp.bfloat16)
    Wob = Wo.astype(jnp.bfloat16)

    def body(x_ref, wq_ref, k_ref, v_ref, wo_ref, o_ref,
             big, tiny, tacc, tsend, trecv, sA, sB, rX, sF, rF1, rF3):
        my = lax.axis_index("i")
        left = (my - 1) % N_DEV
        right = (my + 1) % N_DEV
        koff = my * SKV

        barrier_sem = pltpu.get_barrier_semaphore()
        for nbr in (left, right):
            pl.semaphore_signal(barrier_sem, inc=1, device_id=(nbr,),
                                device_id_type=pl.DeviceIdType.MESH)
        pl.semaphore_wait(barrier_sem, 2)

        def project_q(qstart):
            qb = lax.dot_general(
                x_ref[pl.ds(qstart, QBLK), :], wq_ref[...],
                (((1,), (0,)), ((), ())),
                preferred_element_type=jnp.float32)
            return (qb * SCALE).astype(jnp.bfloat16)

        def full_block(qstart):
            q_blk = project_q(qstart)
            qi = qstart + lax.broadcasted_iota(jnp.int32, (QBLK, SKV), 0)
            ki = koff + lax.broadcasted_iota(jnp.int32, (QBLK, SKV), 1)
            keep = (jnp.abs(qi - ki) <= 128) | (ki < 32) | (qi < 32)
            nums, ls = [], []
            for h in range(HQ):
                hc = slice(h * DH, (h + 1) * DH)
                s = lax.dot_general(q_blk[:, hc], k_ref[:, hc],
                                    (((1,), (1,)), ((), ())),
                                    preferred_element_type=jnp.float32)
                w = jnp.where(keep, jnp.exp(s), 0.0)
                num = lax.dot_general(w.astype(jnp.bfloat16), v_ref[:, hc],
                                      (((1,), (0,)), ((), ())),
                                      preferred_element_type=jnp.float32)
                nums.append(num)
                ls.append(jnp.sum(w, axis=1, keepdims=True))
            return nums, ls

        def pack_l(ls):
            return jnp.concatenate(
                ls + [jnp.zeros((QBLK, 128 - HQ), jnp.float32)], axis=1)

        nums0, ls0 = full_block(0)
        for h in range(HQ):
            tiny[0, 0:32, h * DH:(h + 1) * DH] = (
                nums0[h][0:32].astype(jnp.bfloat16))
        tiny[0, 0:32, DM:PACK] = pack_l(ls0)[0:32].astype(jnp.bfloat16)
        nums15, ls15 = full_block(SQ - QBLK)
        for h in range(HQ):
            tiny[0, 32:TINY, h * DH:(h + 1) * DH] = (
                nums15[h].astype(jnp.bfloat16))
        tiny[0, 32:TINY, DM:PACK] = pack_l(ls15).astype(jnp.bfloat16)
        tacc[...] = tiny[0].astype(jnp.float32)

        def band_block(b, carry):
            qstart = b * QBLK
            sb = jnp.minimum(QBLK * (b - 1), SKV - 3 * QBLK)
            q_blk = project_q(qstart)
            qi = qstart + lax.broadcasted_iota(jnp.int32, (QBLK, SLAB), 0)
            c0 = lax.broadcasted_iota(jnp.int32, (QBLK, QBLK), 1)
            cb = sb + lax.broadcasted_iota(jnp.int32, (QBLK, 3 * QBLK), 1)
            kcols = jnp.concatenate([c0, cb], axis=1)
            keep = (jnp.abs(qi - kcols) <= 128) | (kcols < 32) | (qi < 32)
            seg0 = lax.broadcasted_iota(jnp.int32, (QBLK, SLAB), 1) < QBLK
            keep = keep & jnp.logical_not(seg0 & (sb == 0))
            for h in range(HQ):
                hc = slice(h * DH, (h + 1) * DH)
                ksl = jnp.concatenate(
                    [k_ref[0:QBLK, hc], k_ref[pl.ds(sb, 3 * QBLK), hc]],
                    axis=0)
                vsl = jnp.concatenate(
                    [v_ref[0:QBLK, hc], v_ref[pl.ds(sb, 3 * QBLK), hc]],
                    axis=0)
                s = lax.dot_general(q_blk[:, hc], ksl,
                                    (((1,), (1,)), ((), ())),
                                    preferred_element_type=jnp.float32)
                w = jnp.where(keep, jnp.exp(s), 0.0)
                l = jnp.sum(w, axis=1, keepdims=True)
                num = lax.dot_general(w.astype(jnp.bfloat16), vsl,
                                      (((1,), (0,)), ((), ())),
                                      preferred_element_type=jnp.float32)
                big[pl.ds(qstart, QBLK), hc] = (num / l).astype(jnp.bfloat16)
            return carry

        def chunk_rdma(c, tgt, ssem):
            rows = pl.ds(c * CHUNK, CHUNK)
            return pltpu.make_async_remote_copy(
                src_ref=big.at[rows], dst_ref=big.at[rows],
                send_sem=ssem.at[c], recv_sem=rX.at[c],
                device_id=(tgt,), device_id_type=pl.DeviceIdType.MESH)

        for hop in range(N_DEV - 1):
            ss, rs = hop % 2, (hop + 1) % 2
            rdma_t = pltpu.make_async_remote_copy(
                src_ref=tiny.at[ss], dst_ref=tiny.at[rs],
                send_sem=tsend.at[ss], recv_sem=trecv.at[rs],
                device_id=(right,), device_id_type=pl.DeviceIdType.MESH)
            rdma_t.start()
            lo, hi, chunks = SEGS[hop]

            @pl.when(my == 0)
            def _(hop=hop, lo=lo, hi=hi, chunks=chunks):
                if hop == 0:
                    for h in range(HQ):
                        big[0:QBLK, h * DH:(h + 1) * DH] = (
                            (nums0[h] / ls0[h]).astype(jnp.bfloat16))
                lax.fori_loop(lo, hi, band_block, 0)
                for c in chunks:
                    chunk_rdma(c, 1, sA).start()
                    chunk_rdma(c, 3, sB).start()

            rdma_t.wait()
            tacc[...] = tacc[...] + tiny[rs].astype(jnp.float32)

        @pl.when(my == 0)
        def _():
            for h in range(HQ):
                hc = slice(h * DH, (h + 1) * DH)
                ctx_sp = (tacc[:, hc] / tacc[:, DM + h:DM + h + 1]).astype(
                    jnp.bfloat16)
                big[0:32, hc] = ctx_sp[0:32]
                big[SQ - QBLK:SQ, hc] = ctx_sp[32:TINY]
            chunk_rdma(0, 1, sA).start()
            chunk_rdma(7, 3, sB).start()
            chunk_rdma(7, 1, sA).start()
            chunk_rdma(0, 3, sB).start()
            for c in range(N_CHUNK):
                chunk_rdma(c, 1, sA).wait_send()
                chunk_rdma(c, 3, sB).wait_send()
            o_ref[...] = lax.dot_general(big[...], wo_ref[...],
                                         (((1,), (0,)), ((), ())),
                                         preferred_element_type=jnp.float32)

        def wo_chunk(c):
            rows = pl.ds(c * CHUNK, CHUNK)
            o_ref[rows, :] = lax.dot_general(
                big[rows, :], wo_ref[...], (((1,), (0,)), ((), ())),
                preferred_element_type=jnp.float32)

        @pl.when(my == 1)
        def _():
            for c in ORDER:
                chunk_rdma(c, 0, sA).wait_recv()
                if c < HALF:
                    pltpu.make_async_remote_copy(
                        src_ref=big.at[pl.ds(c * CHUNK, CHUNK)],
                        dst_ref=big.at[pl.ds(c * CHUNK, CHUNK)],
                        send_sem=sF.at[c], recv_sem=rF1.at[c],
                        device_id=(2,),
                        device_id_type=pl.DeviceIdType.MESH).start()
                wo_chunk(c)
            for c in range(HALF):
                pltpu.make_async_remote_copy(
                    src_ref=big.at[pl.ds(c * CHUNK, CHUNK)],
                    dst_ref=big.at[pl.ds(c * CHUNK, CHUNK)],
                    send_sem=sF.at[c], recv_sem=rF1.at[c],
                    device_id=(2,),
                    device_id_type=pl.DeviceIdType.MESH).wait_send()

        @pl.when(my == 3)
        def _():
            for c in ORDER:
                chunk_rdma(c, 0, sB).wait_recv()
                if c >= HALF:
                    pltpu.make_async_remote_copy(
                        src_ref=big.at[pl.ds(c * CHUNK, CHUNK)],
                        dst_ref=big.at[pl.ds(c * CHUNK, CHUNK)],
                        send_sem=sF.at[c - HALF], recv_sem=rF3.at[c - HALF],
                        device_id=(2,),
                        device_id_type=pl.DeviceIdType.MESH).start()
                wo_chunk(c)
            for c in range(HALF):
                pltpu.make_async_remote_copy(
                    src_ref=big.at[pl.ds((c + HALF) * CHUNK, CHUNK)],
                    dst_ref=big.at[pl.ds((c + HALF) * CHUNK, CHUNK)],
                    send_sem=sF.at[c], recv_sem=rF3.at[c],
                    device_id=(2,),
                    device_id_type=pl.DeviceIdType.MESH).wait_send()

        @pl.when(my == 2)
        def _():
            for c in ORDER:
                rows = pl.ds(c * CHUNK, CHUNK)
                src_dev = 1 if c < HALF else 3
                rsem = rF1.at[c] if c < HALF else rF3.at[c - HALF]
                pltpu.make_async_remote_copy(
                    src_ref=big.at[rows], dst_ref=big.at[rows],
                    send_sem=sF.at[c % HALF], recv_sem=rsem,
                    device_id=(src_dev,),
                    device_id_type=pl.DeviceIdType.MESH).wait_recv()
                wo_chunk(c)

    out = pl.pallas_call(
        body,
        out_shape=jax.ShapeDtypeStruct((SQ, DM), jnp.float32),
        in_specs=[pl.BlockSpec(memory_space=pltpu.VMEM)] * 5,
        out_specs=pl.BlockSpec(memory_space=pltpu.VMEM),
        scratch_shapes=[
            pltpu.VMEM((SQ, DM), jnp.bfloat16),
            pltpu.VMEM((2, TINY, PACK), jnp.bfloat16),
            pltpu.VMEM((TINY, PACK), jnp.float32),
            pltpu.SemaphoreType.DMA((2,)),
            pltpu.SemaphoreType.DMA((2,)),
            pltpu.SemaphoreType.DMA((N_CHUNK,)),
            pltpu.SemaphoreType.DMA((N_CHUNK,)),
            pltpu.SemaphoreType.DMA((N_CHUNK,)),
            pltpu.SemaphoreType.DMA((HALF,)),
            pltpu.SemaphoreType.DMA((HALF,)),
            pltpu.SemaphoreType.DMA((HALF,)),
        ],
        compiler_params=pltpu.CompilerParams(
            collective_id=0, vmem_limit_bytes=60 * 1024 * 1024),
    )(xb, Wqb, Kb, Vb, Wob)
    return out.reshape(1, SQ, DM)
